# baseline (speedup 1.0000x reference)
import functools

import jax
import jax.numpy as jnp
import numpy as np
from jax import lax
from jax.experimental import pallas as pl
from jax.experimental.pallas import tpu as pltpu
from jax.experimental.pallas import tpu_sc as plsc

N_STATES = 128
BATCH = 1024
ROW_D = 256
ALPHA = np.float32(0.5)
UNIF_C = np.float32((1.0 - 0.5) / 128)
TINY = np.float32(np.finfo(np.float32).tiny)

_KS0 = np.uint32(0)
_KS1 = np.uint32(42)
_KS2 = np.uint32(0 ^ 42 ^ 0x1BD11BDA)
_ROTS = ((13, 15, 26, 6), (17, 29, 16, 24))


def _rotl(x, r):
    return (x << np.uint32(r)) | (x >> np.uint32(32 - r))


def _threefry_bits(x1):
    ks = (_KS0, _KS1, _KS2)
    x0 = jnp.zeros_like(x1) + ks[0]
    x1 = x1 + ks[1]
    for i in range(5):
        for r in _ROTS[i % 2]:
            x0 = x0 + x1
            x1 = _rotl(x1, r)
            x1 = x0 ^ x1
        x0 = x0 + ks[(i + 1) % 3]
        x1 = x1 + ks[(i + 2) % 3] + np.uint32(i + 1)
    return x0 ^ x1


def _sample_body(p_ref, idx_ref, pn_ref, lt_ref, dt_ref):
    s = pl.program_id(0)

    @pl.when(s == 0)
    def _():
        p = p_ref[...]
        rp = ALPHA * jnp.exp(p) + UNIF_C
        lt = jnp.log(rp)
        lt_ref[...] = lt
        dt_ref[...] = p - lt

    lt = lt_ref[...]
    dt = dt_ref[...]
    iota_k = lax.broadcasted_iota(jnp.int32, (N_STATES, BATCH), 0)
    iota_b = lax.broadcasted_iota(jnp.int32, (N_STATES, BATCH), 1)
    j = (s * (BATCH * N_STATES) + iota_b * N_STATES + iota_k).astype(jnp.uint32)
    bits = _threefry_bits(j)
    fb = (bits >> np.uint32(9)) | np.uint32(0x3F800000)
    f = lax.bitcast_convert_type(fb, jnp.float32) - np.float32(1.0)
    u = jnp.maximum(TINY, f * (np.float32(1.0) - TINY) + TINY)
    g = -jnp.log(-jnp.log(u))
    cand = g + lt
    m = jnp.max(cand, axis=0, keepdims=True)
    idx = jnp.min(
        jnp.where(cand == m, iota_k, np.int32(N_STATES)), axis=0, keepdims=True
    )
    pn = jnp.sum(jnp.where(iota_k == idx, dt, np.float32(0.0)), axis=0, keepdims=True)
    flat = idx * BATCH + iota_b[0:1, :]
    idx_ref[...] = flat.reshape(1, 1, BATCH)
    pn_ref[...] = pn.reshape(1, 1, BATCH)


def _sample_call(p2d):
    return pl.pallas_call(
        _sample_body,
        grid=(N_STATES,),
        in_specs=[pl.BlockSpec((N_STATES, BATCH), lambda s: (0, 0))],
        out_specs=[
            pl.BlockSpec((1, 1, BATCH), lambda s: (s, 0, 0)),
            pl.BlockSpec((1, 1, BATCH), lambda s: (s, 0, 0)),
        ],
        out_shape=[
            jax.ShapeDtypeStruct((N_STATES, 1, BATCH), jnp.int32),
            jax.ShapeDtypeStruct((N_STATES, 1, BATCH), jnp.float32),
        ],
        scratch_shapes=[
            pltpu.VMEM((N_STATES, BATCH), jnp.float32),
            pltpu.VMEM((N_STATES, BATCH), jnp.float32),
        ],
    )(p2d)


def _lse_body(x_ref, o_ref):
    x = x_ref[...]
    m = jnp.max(x, axis=0, keepdims=True)
    lse = jnp.log(jnp.sum(jnp.exp(x - m), axis=0, keepdims=True)) + m
    o_ref[...] = x - lse


def _lse_call(pn2d):
    return pl.pallas_call(
        _lse_body,
        out_shape=jax.ShapeDtypeStruct((N_STATES, BATCH), jnp.float32),
    )(pn2d)


_N_WORKERS = 32
_CHUNK = 128
_ROWS_PER_W = (N_STATES * BATCH) // _N_WORKERS
_CHUNKS_PER_W = _ROWS_PER_W // _CHUNK


def _gather_body(states_hbm, idx_hbm, out_hbm, idx_v, buf, sem):
    info = plsc.get_sparse_core_info()
    nc = info.num_cores
    wid = lax.axis_index("s") * nc + lax.axis_index("c")
    crow0 = wid * _CHUNKS_PER_W
    pltpu.sync_copy(idx_hbm.at[pl.ds(crow0, _CHUNKS_PER_W)], idx_v)

    @pl.loop(0, _CHUNKS_PER_W)
    def _(c):
        pltpu.async_copy(states_hbm.at[idx_v.at[c]], buf, sem).wait()
        row0 = (crow0 + c) * _CHUNK
        pltpu.sync_copy(buf, out_hbm.at[pl.ds(row0, _CHUNK)])


def _gather_call(states, idx2d):
    mesh = plsc.VectorSubcoreMesh(core_axis_name="c", subcore_axis_name="s")
    f = pl.kernel(
        _gather_body,
        out_type=jax.ShapeDtypeStruct((N_STATES * BATCH, ROW_D), jnp.float32),
        mesh=mesh,
        scratch_types=[
            pltpu.VMEM((_CHUNKS_PER_W, _CHUNK), jnp.int32),
            pltpu.VMEM((_CHUNK, ROW_D), jnp.float32),
            pltpu.SemaphoreType.DMA,
        ],
    )
    return f(states, idx2d)


def kernel(states, prob):
    p2d = prob.reshape(N_STATES, BATCH)
    flat3, pn3 = _sample_call(p2d)
    prob_new = _lse_call(pn3.reshape(N_STATES, BATCH)).reshape(-1, 1)
    idx2d = flat3.reshape(-1, _CHUNK)
    new_states = _gather_call(states, idx2d)
    return new_states, prob_new


# speedup vs baseline: 1.3689x; 1.3689x over previous
import functools

import jax
import jax.numpy as jnp
import numpy as np
from jax import lax
from jax.experimental import pallas as pl
from jax.experimental.pallas import tpu as pltpu
from jax.experimental.pallas import tpu_sc as plsc

N_STATES = 128
BATCH = 1024
ROW_D = 256
ALPHA = np.float32(0.5)
UNIF_C = np.float32((1.0 - 0.5) / 128)
TINY = np.float32(np.finfo(np.float32).tiny)

_KS0 = np.uint32(0)
_KS1 = np.uint32(42)
_KS2 = np.uint32(0 ^ 42 ^ 0x1BD11BDA)
_ROTS = ((13, 15, 26, 6), (17, 29, 16, 24))


def _rotl(x, r):
    return (x << np.uint32(r)) | (x >> np.uint32(32 - r))


def _threefry_bits(x1):
    ks = (_KS0, _KS1, _KS2)
    x0 = jnp.zeros_like(x1) + ks[0]
    x1 = x1 + ks[1]
    for i in range(5):
        for r in _ROTS[i % 2]:
            x0 = x0 + x1
            x1 = _rotl(x1, r)
            x1 = x0 ^ x1
        x0 = x0 + ks[(i + 1) % 3]
        x1 = x1 + ks[(i + 2) % 3] + np.uint32(i + 1)
    return x0 ^ x1


_BC = 128


def _sample_body(p_ref, idx_ref, pn_ref, ir_ref, dt_ref):
    s = pl.program_id(0)

    @pl.when(s == 0)
    def _():
        p = p_ref[...]
        rp = ALPHA * jnp.exp(p) + UNIF_C
        ir_ref[...] = np.float32(1.0) / rp
        dt_ref[...] = p - jnp.log(rp)

    iota_k = lax.broadcasted_iota(jnp.int32, (N_STATES, _BC), 0)
    iota_b = lax.broadcasted_iota(jnp.int32, (N_STATES, _BC), 1)
    for bc in range(BATCH // _BC):
        ir = ir_ref[:, pl.ds(bc * _BC, _BC)]
        dt = dt_ref[:, pl.ds(bc * _BC, _BC)]
        j = (
            s * (BATCH * N_STATES) + (bc * _BC + iota_b) * N_STATES + iota_k
        ).astype(jnp.uint32)
        bits = _threefry_bits(j)
        fb = (bits >> np.uint32(9)) | np.uint32(0x3F800000)
        f = lax.bitcast_convert_type(fb, jnp.float32) - np.float32(1.0)
        u = jnp.maximum(TINY, f * (np.float32(1.0) - TINY) + TINY)
        score = -jnp.log(u) * ir
        mn = jnp.min(score, axis=0, keepdims=True)
        idx = jnp.min(
            jnp.where(score == mn, iota_k, np.int32(N_STATES)), axis=0, keepdims=True
        )
        pn = jnp.sum(
            jnp.where(iota_k == idx, dt, np.float32(0.0)), axis=0, keepdims=True
        )
        flat = idx * BATCH + (bc * _BC + iota_b[0:1, :])
        idx_ref[0:1, 0:1, pl.ds(bc * _BC, _BC)] = flat.reshape(1, 1, _BC)
        pn_ref[0:1, 0:1, pl.ds(bc * _BC, _BC)] = pn.reshape(1, 1, _BC)


def _sample_call(p2d):
    return pl.pallas_call(
        _sample_body,
        grid=(N_STATES,),
        in_specs=[pl.BlockSpec((N_STATES, BATCH), lambda s: (0, 0))],
        out_specs=[
            pl.BlockSpec((1, 1, BATCH), lambda s: (s, 0, 0)),
            pl.BlockSpec((1, 1, BATCH), lambda s: (s, 0, 0)),
        ],
        out_shape=[
            jax.ShapeDtypeStruct((N_STATES, 1, BATCH), jnp.int32),
            jax.ShapeDtypeStruct((N_STATES, 1, BATCH), jnp.float32),
        ],
        scratch_shapes=[
            pltpu.VMEM((N_STATES, BATCH), jnp.float32),
            pltpu.VMEM((N_STATES, BATCH), jnp.float32),
        ],
    )(p2d)


def _lse_body(x_ref, o_ref):
    x = x_ref[...]
    m = jnp.max(x, axis=0, keepdims=True)
    lse = jnp.log(jnp.sum(jnp.exp(x - m), axis=0, keepdims=True)) + m
    o_ref[...] = x - lse


def _lse_call(pn2d):
    return pl.pallas_call(
        _lse_body,
        out_shape=jax.ShapeDtypeStruct((N_STATES, BATCH), jnp.float32),
    )(pn2d)


_N_WORKERS = 32
_CHUNK = 128
_ROWS_PER_W = (N_STATES * BATCH) // _N_WORKERS
_CHUNKS_PER_W = _ROWS_PER_W // _CHUNK


def _gather_body(states_hbm, idx_hbm, out_hbm, idx_v, buf, sem):
    info = plsc.get_sparse_core_info()
    nc = info.num_cores
    wid = lax.axis_index("s") * nc + lax.axis_index("c")
    crow0 = wid * _CHUNKS_PER_W
    pltpu.sync_copy(idx_hbm.at[pl.ds(crow0, _CHUNKS_PER_W)], idx_v)

    @pl.loop(0, _CHUNKS_PER_W)
    def _(c):
        pltpu.async_copy(states_hbm.at[idx_v.at[c]], buf, sem).wait()
        row0 = (crow0 + c) * _CHUNK
        pltpu.sync_copy(buf, out_hbm.at[pl.ds(row0, _CHUNK)])


def _gather_call(states, idx2d):
    mesh = plsc.VectorSubcoreMesh(core_axis_name="c", subcore_axis_name="s")
    f = pl.kernel(
        _gather_body,
        out_type=jax.ShapeDtypeStruct((N_STATES * BATCH, ROW_D), jnp.float32),
        mesh=mesh,
        scratch_types=[
            pltpu.VMEM((_CHUNKS_PER_W, _CHUNK), jnp.int32),
            pltpu.VMEM((_CHUNK, ROW_D), jnp.float32),
            pltpu.SemaphoreType.DMA,
        ],
    )
    return f(states, idx2d)


def kernel(states, prob):
    p2d = prob.reshape(N_STATES, BATCH)
    flat3, pn3 = _sample_call(p2d)
    prob_new = _lse_call(pn3.reshape(N_STATES, BATCH)).reshape(-1, 1)
    idx2d = flat3.reshape(-1, _CHUNK)
    new_states = _gather_call(states, idx2d)
    return new_states, prob_new
